# baseline (speedup 1.0000x reference)
import jax
import jax.numpy as jnp
from jax import lax
from jax.experimental import pallas as pl
from jax.experimental.pallas import tpu as pltpu
from jax.experimental.pallas import tpu_sc as plsc

NMOL, MAXAT, NPAIR = 512, 48, 768
NRBF = 128
NSPECIES = 10
BLK = 8
NWORKERS = 32
MPW = NMOL // NWORKERS

_HIGH = jax.lax.Precision.HIGHEST
_BF = jnp.bfloat16
_F32 = jnp.float32


def _mm(a, b):
    return jax.lax.dot_general(a, b, (((1,), (0,)), ((), ())),
                               preferred_element_type=_F32)


def _sc_body(cart_hbm, ai_hbm, aj_hbm, sh_hbm, species_hbm,
             s2_hbm, specj_hbm, tot_hbm,
             cart_v0, ai_v0, aj_v0, sh_v0, spec_v0, s2_v0, sj_v0, tot_v0,
             cart_v1, ai_v1, aj_v1, sh_v1, spec_v1, s2_v1, sj_v1, tot_v1,
             in_sem0, in_sem1, out_sem0, out_sem1):
    wid = lax.axis_index("s") * 2 + lax.axis_index("c")
    base = wid * MPW

    slots = (
        (cart_v0, ai_v0, aj_v0, sh_v0, spec_v0, s2_v0, sj_v0, tot_v0,
         in_sem0, out_sem0),
        (cart_v1, ai_v1, aj_v1, sh_v1, spec_v1, s2_v1, sj_v1, tot_v1,
         in_sem1, out_sem1),
    )
    zf = jnp.zeros((16,), _F32)
    iota3 = jax.lax.iota(jnp.int32, 16) * 3

    def issue_in(m, slot):
        cart_v, ai_v, aj_v, sh_v, spec_v = slot[:5]
        in_sem = slot[8]
        pltpu.async_copy(cart_hbm.at[m], cart_v, in_sem)
        pltpu.async_copy(ai_hbm.at[m], ai_v, in_sem)
        pltpu.async_copy(aj_hbm.at[m], aj_v, in_sem)
        pltpu.async_copy(sh_hbm.at[m], sh_v, in_sem)
        pltpu.async_copy(species_hbm.at[m], spec_v, in_sem)

    def wait_in(m, slot):
        cart_v, ai_v, aj_v, sh_v, spec_v = slot[:5]
        in_sem = slot[8]
        pltpu.make_async_copy(cart_hbm.at[m], cart_v, in_sem).wait()
        pltpu.make_async_copy(ai_hbm.at[m], ai_v, in_sem).wait()
        pltpu.make_async_copy(aj_hbm.at[m], aj_v, in_sem).wait()
        pltpu.make_async_copy(sh_hbm.at[m], sh_v, in_sem).wait()
        pltpu.make_async_copy(species_hbm.at[m], spec_v, in_sem).wait()

    issue_in(base, slots[0])
    issue_in(base + 1, slots[1])

    def per_pair(t2, carry):
        for s in range(2):
            slot = slots[s]
            (cart_v, ai_v, aj_v, sh_v, spec_v, s2_v, sj_v, tot_v,
             in_sem, out_sem) = slot
            m = base + 2 * t2 + s
            wait_in(m, slot)
            for c in range(3 * MAXAT // 16):
                tot_v[pl.ds(c * 16, 16)] = zf
            for p in range(0, NPAIR, 16):
                sl = pl.ds(p, 16)
                ivec = ai_v[sl]
                jvec = aj_v[sl]
                i3 = ivec * 3
                j3 = jvec * 3
                ci0 = plsc.load_gather(cart_v, [i3])
                ci1 = plsc.load_gather(cart_v, [i3 + 1])
                ci2 = plsc.load_gather(cart_v, [i3 + 2])
                cj0 = plsc.load_gather(cart_v, [j3])
                cj1 = plsc.load_gather(cart_v, [j3 + 1])
                cj2 = plsc.load_gather(cart_v, [j3 + 2])
                sb = iota3 + 3 * p
                sh0 = plsc.load_gather(sh_v, [sb])
                sh1 = plsc.load_gather(sh_v, [sb + 1])
                sh2 = plsc.load_gather(sh_v, [sb + 2])
                dv0 = ci0 - cj0 + sh0
                dv1 = ci1 - cj1 + sh1
                dv2 = ci2 - cj2 + sh2
                s2_v[sl] = dv0 * dv0 + dv1 * dv1 + dv2 * dv2
                valid = (sh0 > -1e10) & (sh1 > -1e10) & (sh2 > -1e10)
                plsc.addupdate_scatter(tot_v, [ivec],
                                       jnp.where(valid, dv0, 0.0))
                plsc.addupdate_scatter(tot_v, [ivec + MAXAT],
                                       jnp.where(valid, dv1, 0.0))
                plsc.addupdate_scatter(tot_v, [ivec + 2 * MAXAT],
                                       jnp.where(valid, dv2, 0.0))
                sj_v[sl] = plsc.load_gather(spec_v, [jvec])
            issue_in(jnp.minimum(m + 2, NMOL - 1), slot)
            out_s2 = pltpu.make_async_copy(s2_v, s2_hbm.at[m], out_sem)
            out_sj = pltpu.make_async_copy(sj_v, specj_hbm.at[m], out_sem)
            out_tot = pltpu.make_async_copy(tot_v, tot_hbm.at[m], out_sem)
            out_s2.start()
            out_sj.start()
            out_tot.start()
            out_s2.wait()
            out_sj.wait()
            out_tot.wait()
        return carry

    lax.fori_loop(0, MPW // 2, per_pair, 0)
    wait_in(base, slots[0])
    wait_in(base, slots[1])


def _run_sc(cart_flat, ai, aj, sh_flat, species):
    fn = pl.kernel(
        _sc_body,
        out_type=[
            jax.ShapeDtypeStruct((NMOL, NPAIR), _F32),
            jax.ShapeDtypeStruct((NMOL, NPAIR), jnp.int32),
            jax.ShapeDtypeStruct((NMOL, 3 * MAXAT), _F32),
        ],
        mesh=plsc.VectorSubcoreMesh(core_axis_name="c", subcore_axis_name="s"),
        compiler_params=pltpu.CompilerParams(needs_layout_passes=False),
        scratch_types=(
            [pltpu.VMEM((3 * MAXAT,), _F32),
             pltpu.VMEM((NPAIR,), jnp.int32),
             pltpu.VMEM((NPAIR,), jnp.int32),
             pltpu.VMEM((3 * NPAIR,), _F32),
             pltpu.VMEM((MAXAT,), jnp.int32),
             pltpu.VMEM((NPAIR,), _F32),
             pltpu.VMEM((NPAIR,), jnp.int32),
             pltpu.VMEM((3 * MAXAT,), _F32)] * 2
            + [pltpu.SemaphoreType.DMA] * 4
        ),
    )
    return fn(cart_flat, ai, aj, sh_flat, species)


def _tc_body(s2_ref, specj_ref, tot_ref, idxi_ref, W_embT_ref, W1T_ref,
             b1_ref, W2T_ref, b2_ref, centers_ref, out_ref):
    W_embT = W_embT_ref[...]
    W1T = W1T_ref[...]
    b1 = b1_ref[...]
    W2T = W2T_ref[...]
    b2 = b2_ref[0, 0]
    centers = centers_ref[...]
    neg4c2 = -4.0 * centers * centers

    iota_pa = jax.lax.broadcasted_iota(jnp.int32, (NPAIR, MAXAT), 1)
    iota_sp = jax.lax.broadcasted_iota(jnp.int32, (NSPECIES, NPAIR), 0)

    for b in range(BLK):
        s2row = s2_ref[b:b + 1, :] + 1e-12
        dist = jnp.sqrt(s2row)
        arg = centers * (8.0 * dist) + (neg4c2 + (-4.0) * s2row)
        rbfT = jnp.exp(arg)

        specj = specj_ref[b:b + 1, :]
        SpecJ = (iota_sp == specj).astype(_BF)
        embjT = _mm(W_embT, SpecJ)
        contribT = (rbfT * embjT).astype(_BF)

        Pi = (iota_pa == idxi_ref[b]).astype(_BF)
        densT = _mm(contribT, Pi)
        hT = jnp.tanh(_mm(W1T, densT.astype(_BF)) + b1)
        outT = jnp.dot(W2T, hT, precision=_HIGH) + b2

        totT = tot_ref[b]
        dipoleT = jnp.sum(totT * outT, axis=1, keepdims=True)
        out_ref[0, :, b:b + 1] = dipoleT


def kernel(cart, numatoms, species, atom_index, shifts, W_emb, W1, b1, W2,
           b2, centers):
    del numatoms
    nmol = cart.shape[0]
    cart_flat = cart.reshape(nmol, 3 * MAXAT)
    ai = atom_index[0].astype(jnp.int32)
    aj = atom_index[1].astype(jnp.int32)
    sh_flat = shifts.reshape(nmol, 3 * NPAIR)
    species32 = species.astype(jnp.int32)

    s2, specj, tot_vec = _run_sc(cart_flat, ai, aj, sh_flat, species32)
    tot_vec = tot_vec.reshape(nmol, 3, MAXAT)
    idx_i_col = ai[..., None]

    grid = (nmol // BLK,)
    out = pl.pallas_call(
        _tc_body,
        grid=grid,
        in_specs=[
            pl.BlockSpec((BLK, NPAIR), lambda m: (m, 0)),
            pl.BlockSpec((BLK, NPAIR), lambda m: (m, 0)),
            pl.BlockSpec((BLK, 3, MAXAT), lambda m: (m, 0, 0)),
            pl.BlockSpec((BLK, NPAIR, 1), lambda m: (m, 0, 0)),
            pl.BlockSpec((NRBF, NSPECIES), lambda m: (0, 0)),
            pl.BlockSpec((128, NRBF), lambda m: (0, 0)),
            pl.BlockSpec((128, 1), lambda m: (0, 0)),
            pl.BlockSpec((1, 128), lambda m: (0, 0)),
            pl.BlockSpec((1, 1), lambda m: (0, 0)),
            pl.BlockSpec((NRBF, 1), lambda m: (0, 0)),
        ],
        out_specs=pl.BlockSpec((1, 3, BLK), lambda m: (m, 0, 0)),
        out_shape=jax.ShapeDtypeStruct((nmol // BLK, 3, BLK), jnp.float32),
    )(s2, specj, tot_vec, idx_i_col, W_emb.T.astype(_BF), W1.T.astype(_BF),
      b1.reshape(128, 1), W2.reshape(1, 128), b2.reshape(1, 1),
      centers.reshape(NRBF, 1))
    return (jnp.transpose(out, (0, 2, 1)).reshape(nmol, 3),)


# speedup vs baseline: 1.0391x; 1.0391x over previous
import jax
import jax.numpy as jnp
from jax import lax
from jax.experimental import pallas as pl
from jax.experimental.pallas import tpu as pltpu
from jax.experimental.pallas import tpu_sc as plsc

NMOL, MAXAT, NPAIR = 512, 48, 768
NRBF = 128
NSPECIES = 10
BLK = 16
NWORKERS = 32
MPW = NMOL // NWORKERS

_HIGH = jax.lax.Precision.HIGHEST
_BF = jnp.bfloat16
_F32 = jnp.float32


def _mm(a, b):
    return jax.lax.dot_general(a, b, (((1,), (0,)), ((), ())),
                               preferred_element_type=_F32)


def _sc_body(cart_hbm, ai_hbm, aj_hbm, sh_hbm, species_hbm,
             s2_hbm, specj_hbm, tot_hbm,
             cart_v0, ai_v0, aj_v0, sh_v0, spec_v0, s2_v0, sj_v0, tot_v0,
             cart_v1, ai_v1, aj_v1, sh_v1, spec_v1, s2_v1, sj_v1, tot_v1,
             in_sem0, in_sem1, out_sem0, out_sem1):
    wid = lax.axis_index("s") * 2 + lax.axis_index("c")
    base = wid * MPW

    slots = (
        (cart_v0, ai_v0, aj_v0, sh_v0, spec_v0, s2_v0, sj_v0, tot_v0,
         in_sem0, out_sem0),
        (cart_v1, ai_v1, aj_v1, sh_v1, spec_v1, s2_v1, sj_v1, tot_v1,
         in_sem1, out_sem1),
    )
    zf = jnp.zeros((16,), _F32)
    iota3 = jax.lax.iota(jnp.int32, 16) * 3

    def issue_in(m, slot):
        cart_v, ai_v, aj_v, sh_v, spec_v = slot[:5]
        in_sem = slot[8]
        pltpu.async_copy(cart_hbm.at[m], cart_v, in_sem)
        pltpu.async_copy(ai_hbm.at[m], ai_v, in_sem)
        pltpu.async_copy(aj_hbm.at[m], aj_v, in_sem)
        pltpu.async_copy(sh_hbm.at[m], sh_v, in_sem)
        pltpu.async_copy(species_hbm.at[m], spec_v, in_sem)

    def wait_in(m, slot):
        cart_v, ai_v, aj_v, sh_v, spec_v = slot[:5]
        in_sem = slot[8]
        pltpu.make_async_copy(cart_hbm.at[m], cart_v, in_sem).wait()
        pltpu.make_async_copy(ai_hbm.at[m], ai_v, in_sem).wait()
        pltpu.make_async_copy(aj_hbm.at[m], aj_v, in_sem).wait()
        pltpu.make_async_copy(sh_hbm.at[m], sh_v, in_sem).wait()
        pltpu.make_async_copy(species_hbm.at[m], spec_v, in_sem).wait()

    issue_in(base, slots[0])
    issue_in(base + 1, slots[1])

    def per_pair(t2, carry):
        for s in range(2):
            slot = slots[s]
            (cart_v, ai_v, aj_v, sh_v, spec_v, s2_v, sj_v, tot_v,
             in_sem, out_sem) = slot
            m = base + 2 * t2 + s
            wait_in(m, slot)
            for c in range(3 * MAXAT // 16):
                tot_v[pl.ds(c * 16, 16)] = zf
            for p in range(0, NPAIR, 16):
                sl = pl.ds(p, 16)
                ivec = ai_v[sl]
                jvec = aj_v[sl]
                i3 = ivec * 3
                j3 = jvec * 3
                ci0 = plsc.load_gather(cart_v, [i3])
                ci1 = plsc.load_gather(cart_v, [i3 + 1])
                ci2 = plsc.load_gather(cart_v, [i3 + 2])
                cj0 = plsc.load_gather(cart_v, [j3])
                cj1 = plsc.load_gather(cart_v, [j3 + 1])
                cj2 = plsc.load_gather(cart_v, [j3 + 2])
                sb = iota3 + 3 * p
                sh0 = plsc.load_gather(sh_v, [sb])
                sh1 = plsc.load_gather(sh_v, [sb + 1])
                sh2 = plsc.load_gather(sh_v, [sb + 2])
                dv0 = ci0 - cj0 + sh0
                dv1 = ci1 - cj1 + sh1
                dv2 = ci2 - cj2 + sh2
                s2_v[sl] = dv0 * dv0 + dv1 * dv1 + dv2 * dv2
                valid = (sh0 > -1e10) & (sh1 > -1e10) & (sh2 > -1e10)
                plsc.addupdate_scatter(tot_v, [ivec],
                                       jnp.where(valid, dv0, 0.0))
                plsc.addupdate_scatter(tot_v, [ivec + MAXAT],
                                       jnp.where(valid, dv1, 0.0))
                plsc.addupdate_scatter(tot_v, [ivec + 2 * MAXAT],
                                       jnp.where(valid, dv2, 0.0))
                sj_v[sl] = plsc.load_gather(spec_v, [jvec])
            issue_in(jnp.minimum(m + 2, NMOL - 1), slot)
            out_s2 = pltpu.make_async_copy(s2_v, s2_hbm.at[m], out_sem)
            out_sj = pltpu.make_async_copy(sj_v, specj_hbm.at[m], out_sem)
            out_tot = pltpu.make_async_copy(tot_v, tot_hbm.at[m], out_sem)
            out_s2.start()
            out_sj.start()
            out_tot.start()
            out_s2.wait()
            out_sj.wait()
            out_tot.wait()
        return carry

    lax.fori_loop(0, MPW // 2, per_pair, 0)
    wait_in(base, slots[0])
    wait_in(base, slots[1])


def _run_sc(cart_flat, ai, aj, sh_flat, species):
    fn = pl.kernel(
        _sc_body,
        out_type=[
            jax.ShapeDtypeStruct((NMOL, NPAIR), _F32),
            jax.ShapeDtypeStruct((NMOL, NPAIR), jnp.int32),
            jax.ShapeDtypeStruct((NMOL, 3 * MAXAT), _F32),
        ],
        mesh=plsc.VectorSubcoreMesh(core_axis_name="c", subcore_axis_name="s"),
        compiler_params=pltpu.CompilerParams(needs_layout_passes=False),
        scratch_types=(
            [pltpu.VMEM((3 * MAXAT,), _F32),
             pltpu.VMEM((NPAIR,), jnp.int32),
             pltpu.VMEM((NPAIR,), jnp.int32),
             pltpu.VMEM((3 * NPAIR,), _F32),
             pltpu.VMEM((MAXAT,), jnp.int32),
             pltpu.VMEM((NPAIR,), _F32),
             pltpu.VMEM((NPAIR,), jnp.int32),
             pltpu.VMEM((3 * MAXAT,), _F32)] * 2
            + [pltpu.SemaphoreType.DMA] * 4
        ),
    )
    return fn(cart_flat, ai, aj, sh_flat, species)


def _tc_body(s2_ref, specj_ref, tot_ref, idxi_ref, W_embT_ref, W1T_ref,
             b1_ref, W2T_ref, b2_ref, centers_ref, out_ref):
    W_embT = W_embT_ref[...]
    W1T = W1T_ref[...]
    b1 = b1_ref[...]
    W2T = W2T_ref[...]
    b2 = b2_ref[0, 0]
    centers = centers_ref[...]
    neg4c2 = -4.0 * centers * centers

    iota_pa = jax.lax.broadcasted_iota(jnp.int32, (NPAIR, MAXAT), 1)
    iota_sp = jax.lax.broadcasted_iota(jnp.int32, (NSPECIES, NPAIR), 0)

    for b in range(BLK):
        s2row = s2_ref[b:b + 1, :] + 1e-12
        dist = jnp.sqrt(s2row)
        arg = centers * (8.0 * dist) + (neg4c2 + (-4.0) * s2row)
        rbfT = jnp.exp(arg)

        specj = specj_ref[b:b + 1, :]
        SpecJ = (iota_sp == specj).astype(_BF)
        embjT = _mm(W_embT, SpecJ)
        contribT = (rbfT * embjT).astype(_BF)

        Pi = (iota_pa == idxi_ref[b]).astype(_BF)
        densT = _mm(contribT, Pi)
        hT = jnp.tanh(_mm(W1T, densT.astype(_BF)) + b1)
        outT = jnp.dot(W2T, hT, precision=_HIGH) + b2

        totT = tot_ref[b]
        dipoleT = jnp.sum(totT * outT, axis=1, keepdims=True)
        out_ref[0, :, b:b + 1] = dipoleT


def kernel(cart, numatoms, species, atom_index, shifts, W_emb, W1, b1, W2,
           b2, centers):
    del numatoms
    nmol = cart.shape[0]
    cart_flat = cart.reshape(nmol, 3 * MAXAT)
    ai = atom_index[0].astype(jnp.int32)
    aj = atom_index[1].astype(jnp.int32)
    sh_flat = shifts.reshape(nmol, 3 * NPAIR)
    species32 = species.astype(jnp.int32)

    s2, specj, tot_vec = _run_sc(cart_flat, ai, aj, sh_flat, species32)
    tot_vec = tot_vec.reshape(nmol, 3, MAXAT)
    idx_i_col = ai[..., None]

    grid = (nmol // BLK,)
    out = pl.pallas_call(
        _tc_body,
        grid=grid,
        in_specs=[
            pl.BlockSpec((BLK, NPAIR), lambda m: (m, 0)),
            pl.BlockSpec((BLK, NPAIR), lambda m: (m, 0)),
            pl.BlockSpec((BLK, 3, MAXAT), lambda m: (m, 0, 0)),
            pl.BlockSpec((BLK, NPAIR, 1), lambda m: (m, 0, 0)),
            pl.BlockSpec((NRBF, NSPECIES), lambda m: (0, 0)),
            pl.BlockSpec((128, NRBF), lambda m: (0, 0)),
            pl.BlockSpec((128, 1), lambda m: (0, 0)),
            pl.BlockSpec((1, 128), lambda m: (0, 0)),
            pl.BlockSpec((1, 1), lambda m: (0, 0)),
            pl.BlockSpec((NRBF, 1), lambda m: (0, 0)),
        ],
        out_specs=pl.BlockSpec((1, 3, BLK), lambda m: (m, 0, 0)),
        out_shape=jax.ShapeDtypeStruct((nmol // BLK, 3, BLK), jnp.float32),
    )(s2, specj, tot_vec, idx_i_col, W_emb.T.astype(_BF), W1.T.astype(_BF),
      b1.reshape(128, 1), W2.reshape(1, 128), b2.reshape(1, 1),
      centers.reshape(NRBF, 1))
    return (jnp.transpose(out, (0, 2, 1)).reshape(nmol, 3),)


# speedup vs baseline: 1.4567x; 1.4019x over previous
import jax
import jax.numpy as jnp
from jax import lax
from jax.experimental import pallas as pl
from jax.experimental.pallas import tpu as pltpu
from jax.experimental.pallas import tpu_sc as plsc

NMOL, MAXAT, NPAIR = 512, 48, 768
NRBF = 128
NSPECIES = 10
BLK = 32
PAD = 128
NWORKERS = 32
MPW = NMOL // NWORKERS

_HIGH = jax.lax.Precision.HIGHEST
_BF = jnp.bfloat16
_F32 = jnp.float32


def _mm(a, b):
    return jax.lax.dot_general(a, b, (((1,), (0,)), ((), ())),
                               preferred_element_type=_F32)


def _sc_body(cart_hbm, ai_hbm, aj_hbm, sh_hbm, species_hbm,
             s2_hbm, specj_hbm, tot_hbm,
             cart_v0, ai_v0, aj_v0, sh_v0, spec_v0, s2_v0, sj_v0, tot_v0,
             cart_v1, ai_v1, aj_v1, sh_v1, spec_v1, s2_v1, sj_v1, tot_v1,
             in_sem0, in_sem1, out_sem0, out_sem1):
    wid = lax.axis_index("s") * 2 + lax.axis_index("c")
    base = wid * MPW

    slots = (
        (cart_v0, ai_v0, aj_v0, sh_v0, spec_v0, s2_v0, sj_v0, tot_v0,
         in_sem0, out_sem0),
        (cart_v1, ai_v1, aj_v1, sh_v1, spec_v1, s2_v1, sj_v1, tot_v1,
         in_sem1, out_sem1),
    )
    zf = jnp.zeros((16,), _F32)
    iota3 = jax.lax.iota(jnp.int32, 16) * 3

    def issue_in(m, slot):
        cart_v, ai_v, aj_v, sh_v, spec_v = slot[:5]
        in_sem = slot[8]
        pltpu.async_copy(cart_hbm.at[m], cart_v, in_sem)
        pltpu.async_copy(ai_hbm.at[m], ai_v, in_sem)
        pltpu.async_copy(aj_hbm.at[m], aj_v, in_sem)
        pltpu.async_copy(sh_hbm.at[m], sh_v, in_sem)
        pltpu.async_copy(species_hbm.at[m], spec_v, in_sem)

    def wait_in(m, slot):
        cart_v, ai_v, aj_v, sh_v, spec_v = slot[:5]
        in_sem = slot[8]
        pltpu.make_async_copy(cart_hbm.at[m], cart_v, in_sem).wait()
        pltpu.make_async_copy(ai_hbm.at[m], ai_v, in_sem).wait()
        pltpu.make_async_copy(aj_hbm.at[m], aj_v, in_sem).wait()
        pltpu.make_async_copy(sh_hbm.at[m], sh_v, in_sem).wait()
        pltpu.make_async_copy(species_hbm.at[m], spec_v, in_sem).wait()

    issue_in(base, slots[0])
    issue_in(base + 1, slots[1])

    def per_pair(t2, carry):
        for s in range(2):
            slot = slots[s]
            (cart_v, ai_v, aj_v, sh_v, spec_v, s2_v, sj_v, tot_v,
             in_sem, out_sem) = slot
            m = base + 2 * t2 + s
            wait_in(m, slot)
            for c in range(3 * MAXAT // 16):
                tot_v[pl.ds(c * 16, 16)] = zf
            for p in range(0, NPAIR, 16):
                sl = pl.ds(p, 16)
                ivec = ai_v[sl]
                jvec = aj_v[sl]
                i3 = ivec * 3
                j3 = jvec * 3
                ci0 = plsc.load_gather(cart_v, [i3])
                ci1 = plsc.load_gather(cart_v, [i3 + 1])
                ci2 = plsc.load_gather(cart_v, [i3 + 2])
                cj0 = plsc.load_gather(cart_v, [j3])
                cj1 = plsc.load_gather(cart_v, [j3 + 1])
                cj2 = plsc.load_gather(cart_v, [j3 + 2])
                sb = iota3 + 3 * p
                sh0 = plsc.load_gather(sh_v, [sb])
                sh1 = plsc.load_gather(sh_v, [sb + 1])
                sh2 = plsc.load_gather(sh_v, [sb + 2])
                dv0 = ci0 - cj0 + sh0
                dv1 = ci1 - cj1 + sh1
                dv2 = ci2 - cj2 + sh2
                s2_v[sl] = dv0 * dv0 + dv1 * dv1 + dv2 * dv2
                valid = (sh0 > -1e10) & (sh1 > -1e10) & (sh2 > -1e10)
                plsc.addupdate_scatter(tot_v, [ivec],
                                       jnp.where(valid, dv0, 0.0))
                plsc.addupdate_scatter(tot_v, [ivec + MAXAT],
                                       jnp.where(valid, dv1, 0.0))
                plsc.addupdate_scatter(tot_v, [ivec + 2 * MAXAT],
                                       jnp.where(valid, dv2, 0.0))
                sj_v[sl] = plsc.load_gather(spec_v, [jvec])
            issue_in(jnp.minimum(m + 2, NMOL - 1), slot)
            out_s2 = pltpu.make_async_copy(s2_v, s2_hbm.at[m], out_sem)
            out_sj = pltpu.make_async_copy(sj_v, specj_hbm.at[m], out_sem)
            out_tot = pltpu.make_async_copy(tot_v, tot_hbm.at[m], out_sem)
            out_s2.start()
            out_sj.start()
            out_tot.start()
            out_s2.wait()
            out_sj.wait()
            out_tot.wait()
        return carry

    lax.fori_loop(0, MPW // 2, per_pair, 0)
    wait_in(base, slots[0])
    wait_in(base, slots[1])


def _run_sc(cart_flat, ai, aj, sh_flat, species):
    fn = pl.kernel(
        _sc_body,
        out_type=[
            jax.ShapeDtypeStruct((NMOL, NPAIR), _F32),
            jax.ShapeDtypeStruct((NMOL, NPAIR), jnp.int32),
            jax.ShapeDtypeStruct((NMOL, 3 * MAXAT), _F32),
        ],
        mesh=plsc.VectorSubcoreMesh(core_axis_name="c", subcore_axis_name="s"),
        compiler_params=pltpu.CompilerParams(needs_layout_passes=False),
        scratch_types=(
            [pltpu.VMEM((3 * MAXAT,), _F32),
             pltpu.VMEM((NPAIR,), jnp.int32),
             pltpu.VMEM((NPAIR,), jnp.int32),
             pltpu.VMEM((3 * NPAIR,), _F32),
             pltpu.VMEM((MAXAT,), jnp.int32),
             pltpu.VMEM((NPAIR,), _F32),
             pltpu.VMEM((NPAIR,), jnp.int32),
             pltpu.VMEM((3 * MAXAT,), _F32)] * 2
            + [pltpu.SemaphoreType.DMA] * 4
        ),
    )
    return fn(cart_flat, ai, aj, sh_flat, species)


def _onesb():
    lane = jnp.arange(BLK * PAD, dtype=jnp.int32)
    col = jnp.arange(BLK, dtype=jnp.int32)
    return (((lane[:, None] // PAD) == col[None, :])
            & ((lane[:, None] % PAD) < MAXAT)).astype(_BF)


def _tc_body(s2_ref, specj_ref, tot_ref, idxi_ref, W_embT_ref, W1T_ref,
             b1_ref, W2T_ref, b2_ref, centers_ref, onesb_ref, out_ref,
             dens_all, tot_all):
    W_embT = W_embT_ref[...]
    W1T = W1T_ref[...]
    b1 = b1_ref[...]
    W2T = W2T_ref[...]
    b2 = b2_ref[0, 0]
    centers = centers_ref[...]
    neg4c2 = -4.0 * centers * centers

    iota_pa = jax.lax.broadcasted_iota(jnp.int32, (NPAIR, MAXAT), 1)
    iota_sp = jax.lax.broadcasted_iota(jnp.int32, (NSPECIES, NPAIR), 0)
    zdens = jnp.zeros((NRBF, PAD - MAXAT), _BF)
    ztot = jnp.zeros((3, PAD - MAXAT), _F32)

    for b in range(BLK):
        s2row = s2_ref[b:b + 1, :] + 1e-12
        dist = jnp.sqrt(s2row)
        arg = centers * (8.0 * dist) + (neg4c2 + (-4.0) * s2row)
        rbfT = jnp.exp(arg)

        specj = specj_ref[b:b + 1, :]
        SpecJ = (iota_sp == specj).astype(_BF)
        embjT = _mm(W_embT, SpecJ)
        contribT = (rbfT * embjT).astype(_BF)

        Pi = (iota_pa == idxi_ref[b]).astype(_BF)
        densT = _mm(contribT, Pi)
        dens_all[:, b * PAD:b * PAD + MAXAT] = densT.astype(_BF)
        dens_all[:, b * PAD + MAXAT:(b + 1) * PAD] = zdens
        tot_all[:, b * PAD:b * PAD + MAXAT] = tot_ref[b]
        tot_all[:, b * PAD + MAXAT:(b + 1) * PAD] = ztot

    hT = jnp.tanh(_mm(W1T, dens_all[...]) + b1)
    outT = jnp.dot(W2T, hT, precision=_HIGH) + b2
    prod = tot_all[...] * outT
    prod_hi = prod.astype(_BF)
    prod_lo = (prod - prod_hi.astype(_F32)).astype(_BF)
    onesb = onesb_ref[...]
    dipole = _mm(prod_hi, onesb) + _mm(prod_lo, onesb)
    out_ref[0] = dipole


def kernel(cart, numatoms, species, atom_index, shifts, W_emb, W1, b1, W2,
           b2, centers):
    del numatoms
    nmol = cart.shape[0]
    cart_flat = cart.reshape(nmol, 3 * MAXAT)
    ai = atom_index[0].astype(jnp.int32)
    aj = atom_index[1].astype(jnp.int32)
    sh_flat = shifts.reshape(nmol, 3 * NPAIR)
    species32 = species.astype(jnp.int32)

    s2, specj, tot_vec = _run_sc(cart_flat, ai, aj, sh_flat, species32)
    tot_vec = tot_vec.reshape(nmol, 3, MAXAT)
    idx_i_col = ai[..., None]

    grid = (nmol // BLK,)
    out = pl.pallas_call(
        _tc_body,
        grid=grid,
        in_specs=[
            pl.BlockSpec((BLK, NPAIR), lambda m: (m, 0)),
            pl.BlockSpec((BLK, NPAIR), lambda m: (m, 0)),
            pl.BlockSpec((BLK, 3, MAXAT), lambda m: (m, 0, 0)),
            pl.BlockSpec((BLK, NPAIR, 1), lambda m: (m, 0, 0)),
            pl.BlockSpec((NRBF, NSPECIES), lambda m: (0, 0)),
            pl.BlockSpec((128, NRBF), lambda m: (0, 0)),
            pl.BlockSpec((128, 1), lambda m: (0, 0)),
            pl.BlockSpec((1, 128), lambda m: (0, 0)),
            pl.BlockSpec((1, 1), lambda m: (0, 0)),
            pl.BlockSpec((NRBF, 1), lambda m: (0, 0)),
            pl.BlockSpec((BLK * PAD, BLK), lambda m: (0, 0)),
        ],
        out_specs=pl.BlockSpec((1, 3, BLK), lambda m: (m, 0, 0)),
        out_shape=jax.ShapeDtypeStruct((nmol // BLK, 3, BLK), jnp.float32),
        scratch_shapes=[
            pltpu.VMEM((NRBF, BLK * PAD), _BF),
            pltpu.VMEM((3, BLK * PAD), _F32),
        ],
    )(s2, specj, tot_vec, idx_i_col, W_emb.T.astype(_BF), W1.T.astype(_BF),
      b1.reshape(128, 1), W2.reshape(1, 128), b2.reshape(1, 1),
      centers.reshape(NRBF, 1), _onesb())
    return (jnp.transpose(out, (0, 2, 1)).reshape(nmol, 3),)


# speedup vs baseline: 2.3132x; 1.5880x over previous
import jax
import jax.numpy as jnp
from jax import lax
from jax.experimental import pallas as pl
from jax.experimental.pallas import tpu as pltpu
from jax.experimental.pallas import tpu_sc as plsc

NMOL, MAXAT, NPAIR = 512, 48, 768
NRBF = 128
NSPECIES = 10
BLK = 32
PAD = 128
NWORKERS = 32
MPW = NMOL // NWORKERS

_HIGH = jax.lax.Precision.HIGHEST
_BF = jnp.bfloat16
_F32 = jnp.float32


def _mm(a, b):
    return jax.lax.dot_general(a, b, (((1,), (0,)), ((), ())),
                               preferred_element_type=_F32)


def _sc_body(cart_hbm, ai_hbm, aj_hbm, sh_hbm, species_hbm,
             s2_hbm, specj_hbm, tot_hbm,
             cart_v0, ai_v0, aj_v0, sh_v0, spec_v0, s2_v0, sj_v0, tot_v0,
             cart_v1, ai_v1, aj_v1, sh_v1, spec_v1, s2_v1, sj_v1, tot_v1,
             in_sem0, in_sem1, out_sem0, out_sem1):
    wid = lax.axis_index("s") * 2 + lax.axis_index("c")
    base = wid * MPW

    slots = (
        (cart_v0, ai_v0, aj_v0, sh_v0, spec_v0, s2_v0, sj_v0, tot_v0,
         in_sem0, out_sem0),
        (cart_v1, ai_v1, aj_v1, sh_v1, spec_v1, s2_v1, sj_v1, tot_v1,
         in_sem1, out_sem1),
    )
    zf = jnp.zeros((16,), _F32)
    iota3 = jax.lax.iota(jnp.int32, 16) * 3

    def issue_in(m, slot):
        cart_v, ai_v, aj_v, sh_v, spec_v = slot[:5]
        in_sem = slot[8]
        pltpu.async_copy(cart_hbm.at[m], cart_v, in_sem)
        pltpu.async_copy(ai_hbm.at[m], ai_v, in_sem)
        pltpu.async_copy(aj_hbm.at[m], aj_v, in_sem)
        pltpu.async_copy(sh_hbm.at[m], sh_v, in_sem)
        pltpu.async_copy(species_hbm.at[m], spec_v, in_sem)

    def wait_in(m, slot):
        cart_v, ai_v, aj_v, sh_v, spec_v = slot[:5]
        in_sem = slot[8]
        pltpu.make_async_copy(cart_hbm.at[m], cart_v, in_sem).wait()
        pltpu.make_async_copy(ai_hbm.at[m], ai_v, in_sem).wait()
        pltpu.make_async_copy(aj_hbm.at[m], aj_v, in_sem).wait()
        pltpu.make_async_copy(sh_hbm.at[m], sh_v, in_sem).wait()
        pltpu.make_async_copy(species_hbm.at[m], spec_v, in_sem).wait()

    issue_in(base, slots[0])
    issue_in(base + 1, slots[1])

    def per_pair(t2, carry):
        for s in range(2):
            slot = slots[s]
            (cart_v, ai_v, aj_v, sh_v, spec_v, s2_v, sj_v, tot_v,
             in_sem, out_sem) = slot
            m = base + 2 * t2 + s
            wait_in(m, slot)
            for c in range(3 * MAXAT // 16):
                tot_v[pl.ds(c * 16, 16)] = zf
            for p in range(0, NPAIR, 16):
                sl = pl.ds(p, 16)
                ivec = ai_v[sl]
                jvec = aj_v[sl]
                i3 = ivec * 3
                j3 = jvec * 3
                ci0 = plsc.load_gather(cart_v, [i3])
                ci1 = plsc.load_gather(cart_v, [i3 + 1])
                ci2 = plsc.load_gather(cart_v, [i3 + 2])
                cj0 = plsc.load_gather(cart_v, [j3])
                cj1 = plsc.load_gather(cart_v, [j3 + 1])
                cj2 = plsc.load_gather(cart_v, [j3 + 2])
                sb = iota3 + 3 * p
                sh0 = plsc.load_gather(sh_v, [sb])
                sh1 = plsc.load_gather(sh_v, [sb + 1])
                sh2 = plsc.load_gather(sh_v, [sb + 2])
                dv0 = ci0 - cj0 + sh0
                dv1 = ci1 - cj1 + sh1
                dv2 = ci2 - cj2 + sh2
                s2_v[sl] = dv0 * dv0 + dv1 * dv1 + dv2 * dv2
                valid = (sh0 > -1e10) & (sh1 > -1e10) & (sh2 > -1e10)
                plsc.addupdate_scatter(tot_v, [ivec],
                                       jnp.where(valid, dv0, 0.0))
                plsc.addupdate_scatter(tot_v, [ivec + MAXAT],
                                       jnp.where(valid, dv1, 0.0))
                plsc.addupdate_scatter(tot_v, [ivec + 2 * MAXAT],
                                       jnp.where(valid, dv2, 0.0))
                sj_v[sl] = plsc.load_gather(spec_v, [jvec])
            issue_in(jnp.minimum(m + 2, NMOL - 1), slot)
            out_s2 = pltpu.make_async_copy(s2_v, s2_hbm.at[m], out_sem)
            out_sj = pltpu.make_async_copy(sj_v, specj_hbm.at[m], out_sem)
            out_tot = pltpu.make_async_copy(tot_v, tot_hbm.at[m], out_sem)
            out_s2.start()
            out_sj.start()
            out_tot.start()
            out_s2.wait()
            out_sj.wait()
            out_tot.wait()
        return carry

    lax.fori_loop(0, MPW // 2, per_pair, 0)
    wait_in(base, slots[0])
    wait_in(base, slots[1])


def _run_sc(cart_flat, ai, aj, sh_flat, species):
    fn = pl.kernel(
        _sc_body,
        out_type=[
            jax.ShapeDtypeStruct((NMOL, NPAIR), _F32),
            jax.ShapeDtypeStruct((NMOL, NPAIR), jnp.int32),
            jax.ShapeDtypeStruct((NMOL, 3 * MAXAT), _F32),
        ],
        mesh=plsc.VectorSubcoreMesh(core_axis_name="c", subcore_axis_name="s"),
        compiler_params=pltpu.CompilerParams(needs_layout_passes=False),
        scratch_types=(
            [pltpu.VMEM((3 * MAXAT,), _F32),
             pltpu.VMEM((NPAIR,), jnp.int32),
             pltpu.VMEM((NPAIR,), jnp.int32),
             pltpu.VMEM((3 * NPAIR,), _F32),
             pltpu.VMEM((MAXAT,), jnp.int32),
             pltpu.VMEM((NPAIR,), _F32),
             pltpu.VMEM((NPAIR,), jnp.int32),
             pltpu.VMEM((3 * MAXAT,), _F32)] * 2
            + [pltpu.SemaphoreType.DMA] * 4
        ),
    )
    return fn(cart_flat, ai, aj, sh_flat, species)


def _onesb():
    lane = jnp.arange(BLK * PAD, dtype=jnp.int32)
    col = jnp.arange(BLK, dtype=jnp.int32)
    return (((lane[:, None] // PAD) == col[None, :])
            & ((lane[:, None] % PAD) < MAXAT)).astype(_BF)


def _tc_body(s2_ref, specj_ref, tot_ref, idxi_ref, W_embT_ref, W1T_ref,
             b1_ref, W2T_ref, b2_ref, centers_ref, onesb_ref, out_ref,
             dens_all, tot_all):
    W_embT = W_embT_ref[...]
    W1T = W1T_ref[...]
    b1 = b1_ref[...]
    W2T = W2T_ref[...]
    b2 = b2_ref[0, 0]
    centers = centers_ref[...]
    neg4c2 = -4.0 * centers * centers

    iota_ap = jax.lax.broadcasted_iota(jnp.int32, (MAXAT, NPAIR), 0)
    iota_sp = jax.lax.broadcasted_iota(jnp.int32, (NSPECIES, NPAIR), 0)
    zdens = jnp.zeros((NRBF, PAD - MAXAT), _BF)
    ztot = jnp.zeros((3, PAD - MAXAT), _F32)

    for b in range(BLK):
        s2row = s2_ref[b:b + 1, :] + 1e-12
        dist = jnp.sqrt(s2row)
        arg = centers * (8.0 * dist) + (neg4c2 + (-4.0) * s2row)
        rbfT = jnp.exp(arg)

        specj = specj_ref[b:b + 1, :]
        SpecJ = (iota_sp == specj).astype(_BF)
        embjT = _mm(W_embT, SpecJ)
        contribT = (rbfT * embjT).astype(_BF)

        PiT = (iota_ap == idxi_ref[b:b + 1, :]).astype(_BF)
        densT = jax.lax.dot_general(contribT, PiT,
                                    (((1,), (1,)), ((), ())),
                                    preferred_element_type=_F32)
        dens_all[:, b * PAD:b * PAD + MAXAT] = densT.astype(_BF)
        dens_all[:, b * PAD + MAXAT:(b + 1) * PAD] = zdens
        tot_all[:, b * PAD:b * PAD + MAXAT] = tot_ref[b]
        tot_all[:, b * PAD + MAXAT:(b + 1) * PAD] = ztot

    hT = jnp.tanh(_mm(W1T, dens_all[...]) + b1)
    outT = jnp.dot(W2T, hT, precision=_HIGH) + b2
    prod = tot_all[...] * outT
    prod_hi = prod.astype(_BF)
    prod_lo = (prod - prod_hi.astype(_F32)).astype(_BF)
    onesb = onesb_ref[...]
    dipole = _mm(prod_hi, onesb) + _mm(prod_lo, onesb)
    out_ref[0] = dipole


def kernel(cart, numatoms, species, atom_index, shifts, W_emb, W1, b1, W2,
           b2, centers):
    del numatoms
    nmol = cart.shape[0]
    cart_flat = cart.reshape(nmol, 3 * MAXAT)
    ai = atom_index[0].astype(jnp.int32)
    aj = atom_index[1].astype(jnp.int32)
    sh_flat = shifts.reshape(nmol, 3 * NPAIR)
    species32 = species.astype(jnp.int32)

    s2, specj, tot_vec = _run_sc(cart_flat, ai, aj, sh_flat, species32)
    tot_vec = tot_vec.reshape(nmol, 3, MAXAT)

    grid = (nmol // BLK,)
    out = pl.pallas_call(
        _tc_body,
        grid=grid,
        in_specs=[
            pl.BlockSpec((BLK, NPAIR), lambda m: (m, 0)),
            pl.BlockSpec((BLK, NPAIR), lambda m: (m, 0)),
            pl.BlockSpec((BLK, 3, MAXAT), lambda m: (m, 0, 0)),
            pl.BlockSpec((BLK, NPAIR), lambda m: (m, 0)),
            pl.BlockSpec((NRBF, NSPECIES), lambda m: (0, 0)),
            pl.BlockSpec((128, NRBF), lambda m: (0, 0)),
            pl.BlockSpec((128, 1), lambda m: (0, 0)),
            pl.BlockSpec((1, 128), lambda m: (0, 0)),
            pl.BlockSpec((1, 1), lambda m: (0, 0)),
            pl.BlockSpec((NRBF, 1), lambda m: (0, 0)),
            pl.BlockSpec((BLK * PAD, BLK), lambda m: (0, 0)),
        ],
        out_specs=pl.BlockSpec((1, 3, BLK), lambda m: (m, 0, 0)),
        out_shape=jax.ShapeDtypeStruct((nmol // BLK, 3, BLK), jnp.float32),
        scratch_shapes=[
            pltpu.VMEM((NRBF, BLK * PAD), _BF),
            pltpu.VMEM((3, BLK * PAD), _F32),
        ],
    )(s2, specj, tot_vec, ai, W_emb.T.astype(_BF), W1.T.astype(_BF),
      b1.reshape(128, 1), W2.reshape(1, 128), b2.reshape(1, 1),
      centers.reshape(NRBF, 1), _onesb())
    return (jnp.transpose(out, (0, 2, 1)).reshape(nmol, 3),)


# speedup vs baseline: 2.3263x; 1.0057x over previous
import jax
import jax.numpy as jnp
from jax import lax
from jax.experimental import pallas as pl
from jax.experimental.pallas import tpu as pltpu
from jax.experimental.pallas import tpu_sc as plsc

NMOL, MAXAT, NPAIR = 512, 48, 768
NRBF = 128
NSPECIES = 10
BLK = 32
PAD = 128
NWORKERS = 32
MPW = NMOL // NWORKERS

_HIGH = jax.lax.Precision.HIGHEST
_BF = jnp.bfloat16
_F32 = jnp.float32


def _mm(a, b):
    return jax.lax.dot_general(a, b, (((1,), (0,)), ((), ())),
                               preferred_element_type=_F32)


def _sc_body(cart_hbm, ai_hbm, aj_hbm, sh_hbm, species_hbm,
             s2_hbm, specj_hbm, tot_hbm,
             cart_v0, ai_v0, aj_v0, sh_v0, spec_v0, s2_v0, sj_v0, tot_v0,
             cart_v1, ai_v1, aj_v1, sh_v1, spec_v1, s2_v1, sj_v1, tot_v1,
             in_sem0, in_sem1, out_sem0, out_sem1):
    wid = lax.axis_index("s") * 2 + lax.axis_index("c")
    base = wid * MPW

    slots = (
        (cart_v0, ai_v0, aj_v0, sh_v0, spec_v0, s2_v0, sj_v0, tot_v0,
         in_sem0, out_sem0),
        (cart_v1, ai_v1, aj_v1, sh_v1, spec_v1, s2_v1, sj_v1, tot_v1,
         in_sem1, out_sem1),
    )
    zf = jnp.zeros((16,), _F32)
    iota3 = jax.lax.iota(jnp.int32, 16) * 3

    def issue_in(m, slot):
        cart_v, ai_v, aj_v, sh_v, spec_v = slot[:5]
        in_sem = slot[8]
        pltpu.async_copy(cart_hbm.at[m], cart_v, in_sem)
        pltpu.async_copy(ai_hbm.at[m], ai_v, in_sem)
        pltpu.async_copy(aj_hbm.at[m], aj_v, in_sem)
        pltpu.async_copy(sh_hbm.at[m], sh_v, in_sem)
        pltpu.async_copy(species_hbm.at[m], spec_v, in_sem)

    def wait_in(m, slot):
        cart_v, ai_v, aj_v, sh_v, spec_v = slot[:5]
        in_sem = slot[8]
        pltpu.make_async_copy(cart_hbm.at[m], cart_v, in_sem).wait()
        pltpu.make_async_copy(ai_hbm.at[m], ai_v, in_sem).wait()
        pltpu.make_async_copy(aj_hbm.at[m], aj_v, in_sem).wait()
        pltpu.make_async_copy(sh_hbm.at[m], sh_v, in_sem).wait()
        pltpu.make_async_copy(species_hbm.at[m], spec_v, in_sem).wait()

    issue_in(base, slots[0])
    issue_in(base + 1, slots[1])

    def per_pair(t2, carry):
        for s in range(2):
            slot = slots[s]
            (cart_v, ai_v, aj_v, sh_v, spec_v, s2_v, sj_v, tot_v,
             in_sem, out_sem) = slot
            m = base + 2 * t2 + s
            wait_in(m, slot)
            for c in range(3 * MAXAT // 16):
                tot_v[pl.ds(c * 16, 16)] = zf
            for p in range(0, NPAIR, 16):
                sl = pl.ds(p, 16)
                ivec = ai_v[sl]
                jvec = aj_v[sl]
                i3 = ivec * 3
                j3 = jvec * 3
                ci0 = plsc.load_gather(cart_v, [i3])
                ci1 = plsc.load_gather(cart_v, [i3 + 1])
                ci2 = plsc.load_gather(cart_v, [i3 + 2])
                cj0 = plsc.load_gather(cart_v, [j3])
                cj1 = plsc.load_gather(cart_v, [j3 + 1])
                cj2 = plsc.load_gather(cart_v, [j3 + 2])
                sb = iota3 + 3 * p
                sh0 = plsc.load_gather(sh_v, [sb])
                sh1 = plsc.load_gather(sh_v, [sb + 1])
                sh2 = plsc.load_gather(sh_v, [sb + 2])
                dv0 = ci0 - cj0 + sh0
                dv1 = ci1 - cj1 + sh1
                dv2 = ci2 - cj2 + sh2
                s2_v[sl] = dv0 * dv0 + dv1 * dv1 + dv2 * dv2
                valid = (sh0 > -1e10) & (sh1 > -1e10) & (sh2 > -1e10)
                plsc.addupdate_scatter(tot_v, [ivec],
                                       jnp.where(valid, dv0, 0.0))
                plsc.addupdate_scatter(tot_v, [ivec + MAXAT],
                                       jnp.where(valid, dv1, 0.0))
                plsc.addupdate_scatter(tot_v, [ivec + 2 * MAXAT],
                                       jnp.where(valid, dv2, 0.0))
                sj_v[sl] = plsc.load_gather(spec_v, [jvec])
            issue_in(jnp.minimum(m + 2, NMOL - 1), slot)
            out_s2 = pltpu.make_async_copy(s2_v, s2_hbm.at[m], out_sem)
            out_sj = pltpu.make_async_copy(sj_v, specj_hbm.at[m], out_sem)
            out_tot = pltpu.make_async_copy(tot_v, tot_hbm.at[m], out_sem)
            out_s2.start()
            out_sj.start()
            out_tot.start()
            out_s2.wait()
            out_sj.wait()
            out_tot.wait()
        return carry

    lax.fori_loop(0, MPW // 2, per_pair, 0)
    wait_in(base, slots[0])
    wait_in(base, slots[1])


def _run_sc(cart_flat, ai, aj, sh_flat, species):
    fn = pl.kernel(
        _sc_body,
        out_type=[
            jax.ShapeDtypeStruct((NMOL, NPAIR), _F32),
            jax.ShapeDtypeStruct((NMOL, NPAIR), jnp.int32),
            jax.ShapeDtypeStruct((NMOL, 3 * MAXAT), _F32),
        ],
        mesh=plsc.VectorSubcoreMesh(core_axis_name="c", subcore_axis_name="s"),
        compiler_params=pltpu.CompilerParams(needs_layout_passes=False),
        scratch_types=(
            [pltpu.VMEM((3 * MAXAT,), _F32),
             pltpu.VMEM((NPAIR,), jnp.int32),
             pltpu.VMEM((NPAIR,), jnp.int32),
             pltpu.VMEM((3 * NPAIR,), _F32),
             pltpu.VMEM((MAXAT,), jnp.int32),
             pltpu.VMEM((NPAIR,), _F32),
             pltpu.VMEM((NPAIR,), jnp.int32),
             pltpu.VMEM((3 * MAXAT,), _F32)] * 2
            + [pltpu.SemaphoreType.DMA] * 4
        ),
    )
    return fn(cart_flat, ai, aj, sh_flat, species)


def _onesb():
    lane = jnp.arange(BLK * PAD, dtype=jnp.int32)
    col = jnp.arange(BLK, dtype=jnp.int32)
    return (((lane[:, None] // PAD) == col[None, :])
            & ((lane[:, None] % PAD) < MAXAT)).astype(_BF)


def _tc_body(s2_ref, specj_ref, tot_ref, idxi_ref, W_embT_ref, W1T_ref,
             b1_ref, W2T_ref, b2_ref, centers_ref, onesb_ref, out_ref,
             dens_all, tot_all):
    W_embT = W_embT_ref[...]
    W1T = W1T_ref[...]
    b1 = b1_ref[...]
    W2T = W2T_ref[...]
    b2 = b2_ref[0, 0]
    centers = centers_ref[...]
    log2e = 1.4426950408889634
    c8 = centers * (8.0 * log2e)
    neg4c2 = (-4.0 * log2e) * centers * centers

    iota_ap = jax.lax.broadcasted_iota(jnp.int32, (MAXAT, NPAIR), 0)
    iota_sp = jax.lax.broadcasted_iota(jnp.int32, (NSPECIES, NPAIR), 0)
    zdens = jnp.zeros((NRBF, PAD - MAXAT), _BF)
    ztot = jnp.zeros((3, PAD - MAXAT), _F32)

    for b in range(BLK):
        s2row = s2_ref[b:b + 1, :] + 1e-12
        dist = jnp.sqrt(s2row)
        arg = c8 * dist + (neg4c2 + (-4.0 * log2e) * s2row)
        rbfT = jnp.exp2(arg)

        specj = specj_ref[b:b + 1, :]
        SpecJ = (iota_sp == specj).astype(_BF)
        embjT = _mm(W_embT, SpecJ)
        contribT = (rbfT * embjT).astype(_BF)

        PiT = (iota_ap == idxi_ref[b:b + 1, :]).astype(_BF)
        densT = jax.lax.dot_general(contribT, PiT,
                                    (((1,), (1,)), ((), ())),
                                    preferred_element_type=_F32)
        dens_all[:, b * PAD:b * PAD + MAXAT] = densT.astype(_BF)
        dens_all[:, b * PAD + MAXAT:(b + 1) * PAD] = zdens
        tot_all[:, b * PAD:b * PAD + MAXAT] = tot_ref[b]
        tot_all[:, b * PAD + MAXAT:(b + 1) * PAD] = ztot

    hT = jnp.tanh(_mm(W1T, dens_all[...]) + b1)
    outT = jnp.dot(W2T, hT, precision=_HIGH) + b2
    prod = tot_all[...] * outT
    prod_hi = prod.astype(_BF)
    prod_lo = (prod - prod_hi.astype(_F32)).astype(_BF)
    onesb = onesb_ref[...]
    dipole = _mm(prod_hi, onesb) + _mm(prod_lo, onesb)
    out_ref[0] = dipole


def kernel(cart, numatoms, species, atom_index, shifts, W_emb, W1, b1, W2,
           b2, centers):
    del numatoms
    nmol = cart.shape[0]
    cart_flat = cart.reshape(nmol, 3 * MAXAT)
    ai = atom_index[0].astype(jnp.int32)
    aj = atom_index[1].astype(jnp.int32)
    sh_flat = shifts.reshape(nmol, 3 * NPAIR)
    species32 = species.astype(jnp.int32)

    s2, specj, tot_vec = _run_sc(cart_flat, ai, aj, sh_flat, species32)
    tot_vec = tot_vec.reshape(nmol, 3, MAXAT)

    grid = (nmol // BLK,)
    out = pl.pallas_call(
        _tc_body,
        grid=grid,
        in_specs=[
            pl.BlockSpec((BLK, NPAIR), lambda m: (m, 0)),
            pl.BlockSpec((BLK, NPAIR), lambda m: (m, 0)),
            pl.BlockSpec((BLK, 3, MAXAT), lambda m: (m, 0, 0)),
            pl.BlockSpec((BLK, NPAIR), lambda m: (m, 0)),
            pl.BlockSpec((NRBF, NSPECIES), lambda m: (0, 0)),
            pl.BlockSpec((128, NRBF), lambda m: (0, 0)),
            pl.BlockSpec((128, 1), lambda m: (0, 0)),
            pl.BlockSpec((1, 128), lambda m: (0, 0)),
            pl.BlockSpec((1, 1), lambda m: (0, 0)),
            pl.BlockSpec((NRBF, 1), lambda m: (0, 0)),
            pl.BlockSpec((BLK * PAD, BLK), lambda m: (0, 0)),
        ],
        out_specs=pl.BlockSpec((1, 3, BLK), lambda m: (m, 0, 0)),
        out_shape=jax.ShapeDtypeStruct((nmol // BLK, 3, BLK), jnp.float32),
        scratch_shapes=[
            pltpu.VMEM((NRBF, BLK * PAD), _BF),
            pltpu.VMEM((3, BLK * PAD), _F32),
        ],
    )(s2, specj, tot_vec, ai, W_emb.T.astype(_BF), W1.T.astype(_BF),
      b1.reshape(128, 1), W2.reshape(1, 128), b2.reshape(1, 1),
      centers.reshape(NRBF, 1), _onesb())
    return (jnp.transpose(out, (0, 2, 1)).reshape(nmol, 3),)
